# baseline (speedup 1.0000x reference)
import functools

import jax
import jax.numpy as jnp
import numpy as np
from jax import lax
from jax.experimental import pallas as pl
from jax.experimental.pallas import tpu as pltpu
from jax.experimental.pallas import tpu_sc as plsc

_L = 16
_R = 128
_CHK = 128


def _dataset_params_sc(n_rows, batch):
    nw = 32
    bpw = batch // nw
    ch = bpw // _L
    nblk = bpw // _CHK

    mesh = plsc.VectorSubcoreMesh(core_axis_name="c", subcore_axis_name="s")

    @functools.partial(
        pl.kernel,
        mesh=mesh,
        out_type=(
            jax.ShapeDtypeStruct((batch,), jnp.float32),
            jax.ShapeDtypeStruct((batch,), jnp.float32),
            jax.ShapeDtypeStruct((batch,), jnp.float32),
        ),
        scratch_types=[
            pltpu.VMEM((bpw,), jnp.int32),
            pltpu.VMEM((bpw,), jnp.int32),
            pltpu.VMEM((bpw,), jnp.int32),
            pltpu.VMEM((bpw,), jnp.float32),
            pltpu.VMEM((_CHK, 2, _R), jnp.float32),
            pltpu.VMEM((_CHK, 2, _R), jnp.float32),
            pltpu.VMEM((bpw,), jnp.float32),
            pltpu.VMEM((bpw,), jnp.int32),
            pltpu.VMEM((bpw,), jnp.float32),
            pltpu.VMEM((bpw,), jnp.float32),
            pltpu.VMEM((_L,), jnp.int32),
            pltpu.SemaphoreType.DMA,
            pltpu.SemaphoreType.DMA,
            pltpu.SemaphoreType.DMA,
        ],
        compiler_params=pltpu.CompilerParams(needs_layout_passes=False),
    )
    def k(ind_hbm, xt_hbm, xs_hbm, lin_hbm, out0_hbm, out1_hbm, outs_hbm,
          ind_v, tid_v, lane_v, sign_v, bta_v, btb_v, s_v, idx_v,
          o0_v, o1_v, lin_v,
          semta, semtb, sems):
        wid = lax.axis_index("s") * 2 + lax.axis_index("c")
        base = wid * bpw

        pltpu.sync_copy(ind_hbm.at[pl.ds(base, bpw)], ind_v)
        pltpu.sync_copy(lin_hbm, lin_v)

        def stage(j, carry):
            sl = pl.ds(j * _L, _L)
            v = ind_v[sl]
            w = v >= n_rows
            idx = jnp.where(w, v - n_rows, v)
            idx_v[sl] = idx
            tid_v[sl] = lax.shift_right_logical(idx, 7)
            lane_v[sl] = idx & (_R - 1)
            sign_v[sl] = jnp.where(w, jnp.float32(-1.0), jnp.float32(1.0))
            return carry

        lax.fori_loop(0, ch, stage, 0)

        cS = pltpu.async_copy(xs_hbm.at[idx_v], s_v, sems)
        bufs = [(bta_v, semta), (btb_v, semtb)]

        def issue(blk):
            bt, st = bufs[blk % 2]
            tslice = tid_v.at[pl.ds(blk * _CHK, _CHK)]
            return pltpu.async_copy(xt_hbm.at[tslice], bt, st)

        pending = {0: issue(0)}
        for blk in range(nblk):
            if blk + 1 < nblk:
                pending[blk + 1] = issue(blk + 1)
            pending.pop(blk).wait()
            bt_v, _ = bufs[blk % 2]
            boff = blk * _CHK

            def extract(j, carry, bt_v=bt_v, boff=boff):
                gsl = pl.ds(boff + j * _L, _L)
                rows = lin_v[...] + j * _L
                zeros = lin_v[...] * 0
                lanes = lane_v[gsl]
                t0 = plsc.load_gather(bt_v, [rows, zeros, lanes])
                t1 = plsc.load_gather(bt_v, [rows, zeros + 1, lanes])
                o0_v[gsl] = t0 * sign_v[gsl]
                o1_v[gsl] = t1
                return carry

            lax.fori_loop(0, _CHK // _L, extract, 0)

        pltpu.sync_copy(o0_v, out0_hbm.at[pl.ds(base, bpw)])
        pltpu.sync_copy(o1_v, out1_hbm.at[pl.ds(base, bpw)])
        cS.wait()
        pltpu.sync_copy(s_v, outs_hbm.at[pl.ds(base, bpw)])

    return k


def kernel(indices, ds_translation, ds_scale):
    n_rows = ds_translation.shape[0]
    batch = indices.shape[0]
    n_tiles = -(-n_rows // _R)
    pad_rows = n_tiles * _R - n_rows

    xt = jnp.pad(ds_translation, ((0, pad_rows), (0, 0))) \
        .reshape(n_tiles, _R, 2).transpose(0, 2, 1)
    xs = ds_scale[:, 0]
    lin = jnp.asarray(np.arange(_L), jnp.int32)

    k = _dataset_params_sc(n_rows, batch)
    t0s, t1, s = k(indices.astype(jnp.int32), xt, xs, lin)

    translation_delta = jnp.stack([t0s, t1, jnp.zeros_like(t0s)], axis=1)
    return (translation_delta, s.reshape(batch, 1))


# speedup vs baseline: 1.6909x; 1.6909x over previous
import functools

import jax
import jax.numpy as jnp
import numpy as np
from jax import lax
from jax.experimental import pallas as pl
from jax.experimental.pallas import tpu as pltpu
from jax.experimental.pallas import tpu_sc as plsc

_L = 16
_R = 128
_CHK = 128


def _dataset_params_sc(n_rows, batch):
    nw = 32
    bpw = batch // nw
    ch = bpw // _L
    nblk = bpw // _CHK

    mesh = plsc.VectorSubcoreMesh(core_axis_name="c", subcore_axis_name="s")

    @functools.partial(
        pl.kernel,
        mesh=mesh,
        out_type=(
            jax.ShapeDtypeStruct((batch,), jnp.float32),
            jax.ShapeDtypeStruct((batch,), jnp.float32),
            jax.ShapeDtypeStruct((batch,), jnp.float32),
        ),
        scratch_types=[
            pltpu.VMEM((bpw,), jnp.int32),
            pltpu.VMEM((bpw,), jnp.int32),
            pltpu.VMEM((bpw,), jnp.int32),
            pltpu.VMEM((bpw,), jnp.float32),
            pltpu.VMEM((_CHK, 2, _R), jnp.float32),
            pltpu.VMEM((_CHK, 2, _R), jnp.float32),
            pltpu.VMEM((_CHK, 1, _R), jnp.float32),
            pltpu.VMEM((_CHK, 1, _R), jnp.float32),
            pltpu.VMEM((bpw,), jnp.float32),
            pltpu.VMEM((bpw,), jnp.float32),
            pltpu.VMEM((bpw,), jnp.float32),
            pltpu.VMEM((_L,), jnp.int32),
            pltpu.SemaphoreType.DMA,
            pltpu.SemaphoreType.DMA,
            pltpu.SemaphoreType.DMA,
            pltpu.SemaphoreType.DMA,
        ],
        compiler_params=pltpu.CompilerParams(needs_layout_passes=False),
    )
    def k(ind_hbm, xt_hbm, xs_hbm, lin_hbm, out0_hbm, out1_hbm, outs_hbm,
          ind_v, tid_v, lane_v, sign_v, bta_v, btb_v, bsa_v, bsb_v,
          o0_v, o1_v, os_v, lin_v,
          semta, semtb, semsa, semsb):
        wid = lax.axis_index("s") * 2 + lax.axis_index("c")
        base = wid * bpw

        pltpu.sync_copy(ind_hbm.at[pl.ds(base, bpw)], ind_v)
        pltpu.sync_copy(lin_hbm, lin_v)

        def stage(j, carry):
            sl = pl.ds(j * _L, _L)
            v = ind_v[sl]
            w = v >= n_rows
            idx = jnp.where(w, v - n_rows, v)
            tid_v[sl] = lax.shift_right_logical(idx, 7)
            lane_v[sl] = idx & (_R - 1)
            sign_v[sl] = jnp.where(w, jnp.float32(-1.0), jnp.float32(1.0))
            return carry

        lax.fori_loop(0, ch, stage, 0)

        bufs = [(bta_v, bsa_v, semta, semsa), (btb_v, bsb_v, semtb, semsb)]

        def issue(blk):
            bt, bs, st, ss = bufs[blk % 2]
            tslice = tid_v.at[pl.ds(blk * _CHK, _CHK)]
            return (pltpu.async_copy(xt_hbm.at[tslice], bt, st),
                    pltpu.async_copy(xs_hbm.at[tslice], bs, ss))

        pending = {0: issue(0)}
        for blk in range(nblk):
            if blk + 1 < nblk:
                pending[blk + 1] = issue(blk + 1)
            ct, cs = pending.pop(blk)
            ct.wait()
            cs.wait()
            bt_v, bs_v, _, _ = bufs[blk % 2]
            boff = blk * _CHK

            def extract(j, carry, bt_v=bt_v, bs_v=bs_v, boff=boff):
                gsl = pl.ds(boff + j * _L, _L)
                rows = lin_v[...] + j * _L
                zeros = lin_v[...] * 0
                lanes = lane_v[gsl]
                t0 = plsc.load_gather(bt_v, [rows, zeros, lanes])
                t1 = plsc.load_gather(bt_v, [rows, zeros + 1, lanes])
                sv = plsc.load_gather(bs_v, [rows, zeros, lanes])
                o0_v[gsl] = t0 * sign_v[gsl]
                o1_v[gsl] = t1
                os_v[gsl] = sv
                return carry

            lax.fori_loop(0, _CHK // _L, extract, 0)

        pltpu.sync_copy(o0_v, out0_hbm.at[pl.ds(base, bpw)])
        pltpu.sync_copy(o1_v, out1_hbm.at[pl.ds(base, bpw)])
        pltpu.sync_copy(os_v, outs_hbm.at[pl.ds(base, bpw)])

    return k


def kernel(indices, ds_translation, ds_scale):
    n_rows = ds_translation.shape[0]
    batch = indices.shape[0]
    n_tiles = -(-n_rows // _R)
    pad_rows = n_tiles * _R - n_rows

    xt = jnp.pad(ds_translation, ((0, pad_rows), (0, 0))) \
        .reshape(n_tiles, _R, 2).transpose(0, 2, 1)
    xs = jnp.pad(ds_scale, ((0, pad_rows), (0, 0))) \
        .reshape(n_tiles, _R, 1).transpose(0, 2, 1)
    lin = jnp.asarray(np.arange(_L), jnp.int32)

    k = _dataset_params_sc(n_rows, batch)
    t0s, t1, s = k(indices.astype(jnp.int32), xt, xs, lin)

    translation_delta = jnp.stack([t0s, t1, jnp.zeros_like(t0s)], axis=1)
    return (translation_delta, s.reshape(batch, 1))


# speedup vs baseline: 1.6936x; 1.0016x over previous
import functools

import jax
import jax.numpy as jnp
import numpy as np
from jax import lax
from jax.experimental import pallas as pl
from jax.experimental.pallas import tpu as pltpu
from jax.experimental.pallas import tpu_sc as plsc

_L = 16
_R = 128
_CHK = 128


def _dataset_params_sc(n_rows, batch):
    nw = 32
    bpw = batch // nw
    ch = bpw // _L
    nblk = bpw // _CHK

    mesh = plsc.VectorSubcoreMesh(core_axis_name="c", subcore_axis_name="s")

    @functools.partial(
        pl.kernel,
        mesh=mesh,
        out_type=(
            jax.ShapeDtypeStruct((batch,), jnp.float32),
            jax.ShapeDtypeStruct((batch,), jnp.float32),
            jax.ShapeDtypeStruct((batch,), jnp.float32),
        ),
        scratch_types=[
            pltpu.VMEM((bpw,), jnp.int32),
            pltpu.VMEM((bpw,), jnp.int32),
            pltpu.VMEM((bpw,), jnp.int32),
            pltpu.VMEM((bpw,), jnp.float32),
            pltpu.VMEM((_CHK, 2, _R), jnp.float32),
            pltpu.VMEM((_CHK, 2, _R), jnp.float32),
            pltpu.VMEM((_CHK, 1, _R), jnp.float32),
            pltpu.VMEM((_CHK, 1, _R), jnp.float32),
            pltpu.VMEM((bpw,), jnp.float32),
            pltpu.VMEM((bpw,), jnp.float32),
            pltpu.VMEM((bpw,), jnp.float32),
            pltpu.VMEM((_L,), jnp.int32),
            pltpu.SemaphoreType.DMA,
            pltpu.SemaphoreType.DMA,
            pltpu.SemaphoreType.DMA,
            pltpu.SemaphoreType.DMA,
        ],
        compiler_params=pltpu.CompilerParams(needs_layout_passes=False),
    )
    def k(ind_hbm, xt_hbm, xs_hbm, lin_hbm, out0_hbm, out1_hbm, outs_hbm,
          ind_v, tid_v, lane_v, sign_v, bta_v, btb_v, bsa_v, bsb_v,
          o0_v, o1_v, os_v, lin_v,
          semta, semtb, semsa, semsb):
        wid = lax.axis_index("s") * 2 + lax.axis_index("c")
        base = wid * bpw

        pltpu.sync_copy(ind_hbm.at[pl.ds(base, bpw)], ind_v)
        pltpu.sync_copy(lin_hbm, lin_v)

        def stage(j, carry):
            sl = pl.ds(j * _L, _L)
            v = ind_v[sl]
            w = v >= n_rows
            idx = jnp.where(w, v - n_rows, v)
            tid_v[sl] = lax.shift_right_logical(idx, 7)
            lane_v[sl] = idx & (_R - 1)
            sign_v[sl] = jnp.where(w, jnp.float32(-1.0), jnp.float32(1.0))
            return carry

        lax.fori_loop(0, ch, stage, 0)

        bufs = [(bta_v, bsa_v, semta, semsa), (btb_v, bsb_v, semtb, semsb)]

        def issue(blk):
            bt, bs, st, ss = bufs[blk % 2]
            tslice = tid_v.at[pl.ds(blk * _CHK, _CHK)]
            return (pltpu.async_copy(xt_hbm.at[tslice], bt, st),
                    pltpu.async_copy(xs_hbm.at[tslice], bs, ss))

        pending = {0: issue(0)}
        for blk in range(nblk):
            if blk + 1 < nblk:
                pending[blk + 1] = issue(blk + 1)
            ct, cs = pending.pop(blk)
            ct.wait()
            cs.wait()
            bt_v, bs_v, _, _ = bufs[blk % 2]
            boff = blk * _CHK

            def extract(j, carry, bt_v=bt_v, bs_v=bs_v, boff=boff):
                gsl = pl.ds(boff + j * _L, _L)
                rows = lin_v[...] + j * _L
                zeros = lin_v[...] * 0
                lanes = lane_v[gsl]
                t0 = plsc.load_gather(bt_v, [rows, zeros, lanes])
                t1 = plsc.load_gather(bt_v, [rows, zeros + 1, lanes])
                sv = plsc.load_gather(bs_v, [rows, zeros, lanes])
                o0_v[gsl] = t0 * sign_v[gsl]
                o1_v[gsl] = t1
                os_v[gsl] = sv
                return carry

            lax.fori_loop(0, _CHK // _L, extract, 0)

        pltpu.sync_copy(o0_v, out0_hbm.at[pl.ds(base, bpw)])
        pltpu.sync_copy(o1_v, out1_hbm.at[pl.ds(base, bpw)])
        pltpu.sync_copy(os_v, outs_hbm.at[pl.ds(base, bpw)])

    return k


def kernel(indices, ds_translation, ds_scale):
    n_rows = ds_translation.shape[0]
    batch = indices.shape[0]
    n_tiles = -(-n_rows // _R)
    pad_rows = n_tiles * _R - n_rows

    xt = jnp.concatenate(
        [ds_translation, jnp.zeros((pad_rows, 2), jnp.float32)]) \
        .reshape(n_tiles, _R, 2).transpose(0, 2, 1)
    xs = jnp.concatenate(
        [ds_scale, jnp.zeros((pad_rows, 1), jnp.float32)]) \
        .reshape(n_tiles, _R, 1).transpose(0, 2, 1)
    lin = jnp.asarray(np.arange(_L), jnp.int32)

    k = _dataset_params_sc(n_rows, batch)
    t0s, t1, s = k(indices.astype(jnp.int32), xt, xs, lin)

    translation_delta = jnp.stack([t0s, t1, jnp.zeros_like(t0s)], axis=1)
    return (translation_delta, s.reshape(batch, 1))


# speedup vs baseline: 1.6964x; 1.0017x over previous
import functools

import jax
import jax.numpy as jnp
import numpy as np
from jax import lax
from jax.experimental import pallas as pl
from jax.experimental.pallas import tpu as pltpu
from jax.experimental.pallas import tpu_sc as plsc

_L = 16
_R = 128
_CHK = 128


def _dataset_params_sc(n_rows, batch):
    nw = 32
    bpw = batch // nw
    ch = bpw // _L
    nblk = bpw // _CHK

    mesh = plsc.VectorSubcoreMesh(core_axis_name="c", subcore_axis_name="s")

    @functools.partial(
        pl.kernel,
        mesh=mesh,
        out_type=(
            jax.ShapeDtypeStruct((batch,), jnp.float32),
            jax.ShapeDtypeStruct((batch,), jnp.float32),
            jax.ShapeDtypeStruct((batch,), jnp.float32),
        ),
        scratch_types=[
            pltpu.VMEM((bpw,), jnp.int32),
            pltpu.VMEM((bpw,), jnp.int32),
            pltpu.VMEM((bpw,), jnp.int32),
            pltpu.VMEM((bpw,), jnp.float32),
            pltpu.VMEM((_CHK, 2, _R), jnp.float32),
            pltpu.VMEM((_CHK, 2, _R), jnp.float32),
            pltpu.VMEM((_CHK, 1, _R), jnp.float32),
            pltpu.VMEM((_CHK, 1, _R), jnp.float32),
            pltpu.VMEM((bpw,), jnp.float32),
            pltpu.VMEM((bpw,), jnp.float32),
            pltpu.VMEM((bpw,), jnp.float32),
            pltpu.VMEM((_L,), jnp.int32),
            pltpu.SemaphoreType.DMA,
            pltpu.SemaphoreType.DMA,
            pltpu.SemaphoreType.DMA,
            pltpu.SemaphoreType.DMA,
        ],
        compiler_params=pltpu.CompilerParams(needs_layout_passes=False),
    )
    def k(ind_hbm, xt_hbm, xs_hbm, lin_hbm, out0_hbm, out1_hbm, outs_hbm,
          ind_v, tid_v, lane_v, sign_v, bta_v, btb_v, bsa_v, bsb_v,
          o0_v, o1_v, os_v, lin_v,
          semta, semtb, semsa, semsb):
        wid = lax.axis_index("s") * 2 + lax.axis_index("c")
        base = wid * bpw

        pltpu.sync_copy(ind_hbm.at[pl.ds(base, bpw)], ind_v)
        pltpu.sync_copy(lin_hbm, lin_v)

        def stage(j, carry):
            sl = pl.ds(j * _L, _L)
            v = ind_v[sl]
            w = v >= n_rows
            idx = jnp.where(w, v - n_rows, v)
            tid_v[sl] = lax.shift_right_logical(idx, 7)
            lane_v[sl] = idx & (_R - 1)
            sign_v[sl] = jnp.where(w, jnp.float32(-1.0), jnp.float32(1.0))
            return carry

        lax.fori_loop(0, ch, stage, 0)

        bufs = [(bta_v, bsa_v, semta, semsa), (btb_v, bsb_v, semtb, semsb)]

        def issue(blk):
            bt, bs, st, ss = bufs[blk % 2]
            tslice = tid_v.at[pl.ds(blk * _CHK, _CHK)]
            return (pltpu.async_copy(xt_hbm.at[tslice], bt, st),
                    pltpu.async_copy(xs_hbm.at[tslice], bs, ss))

        pending = {0: issue(0)}
        for blk in range(nblk):
            if blk + 1 < nblk:
                pending[blk + 1] = issue(blk + 1)
            ct, cs = pending.pop(blk)
            ct.wait()
            cs.wait()
            bt_v, bs_v, _, _ = bufs[blk % 2]
            boff = blk * _CHK

            def extract(j, carry, bt_v=bt_v, bs_v=bs_v, boff=boff):
                gsl = pl.ds(boff + j * _L, _L)
                rows = lin_v[...] + j * _L
                zeros = lin_v[...] * 0
                lanes = lane_v[gsl]
                t0 = plsc.load_gather(bt_v, [rows, zeros, lanes])
                t1 = plsc.load_gather(bt_v, [rows, zeros + 1, lanes])
                sv = plsc.load_gather(bs_v, [rows, zeros, lanes])
                o0_v[gsl] = t0 * sign_v[gsl]
                o1_v[gsl] = t1
                os_v[gsl] = sv
                return carry

            lax.fori_loop(0, _CHK // _L, extract, 0)

        pltpu.sync_copy(o0_v, out0_hbm.at[pl.ds(base, bpw)])
        pltpu.sync_copy(o1_v, out1_hbm.at[pl.ds(base, bpw)])
        pltpu.sync_copy(os_v, outs_hbm.at[pl.ds(base, bpw)])

    return k


def kernel(indices, ds_translation, ds_scale):
    n_rows = ds_translation.shape[0]
    batch = indices.shape[0]
    n_tiles = -(-n_rows // _R)
    pad_rows = n_tiles * _R - n_rows

    xt = jnp.pad(ds_translation, ((0, pad_rows), (0, 0))) \
        .reshape(n_tiles, _R, 2).transpose(0, 2, 1)
    xs = jnp.pad(ds_scale, ((0, pad_rows), (0, 0))) \
        .reshape(n_tiles, _R, 1).transpose(0, 2, 1)
    lin = jnp.asarray(np.arange(_L), jnp.int32)

    k = _dataset_params_sc(n_rows, batch)
    t0s, t1, s = k(indices.astype(jnp.int32), xt, xs, lin)

    translation_delta = jnp.stack([t0s, t1, jnp.zeros_like(t0s)], axis=1)
    return (translation_delta, s.reshape(batch, 1))


# speedup vs baseline: 2.1983x; 1.2958x over previous
import functools

import jax
import jax.numpy as jnp
from jax import lax
from jax.experimental import pallas as pl
from jax.experimental.pallas import tpu as pltpu
from jax.experimental.pallas import tpu_sc as plsc

_L = 16
_R = 128


def _dataset_params_sc(n_rows, batch):
    nw = 32
    bpw = batch // nw
    ch = bpw // _L

    mesh = plsc.VectorSubcoreMesh(core_axis_name="c", subcore_axis_name="s")

    @functools.partial(
        pl.kernel,
        mesh=mesh,
        out_type=(
            jax.ShapeDtypeStruct((batch,), jnp.float32),
            jax.ShapeDtypeStruct((batch,), jnp.float32),
            jax.ShapeDtypeStruct((batch,), jnp.float32),
        ),
        scratch_types=[
            pltpu.VMEM((bpw,), jnp.int32),
            pltpu.VMEM((bpw,), jnp.int32),
            pltpu.VMEM((bpw,), jnp.int32),
            pltpu.VMEM((bpw,), jnp.int32),
            pltpu.VMEM((bpw,), jnp.float32),
            pltpu.VMEM((bpw,), jnp.float32),
            pltpu.VMEM((bpw,), jnp.float32),
            pltpu.VMEM((bpw,), jnp.float32),
            pltpu.SemaphoreType.DMA,
            pltpu.SemaphoreType.DMA,
            pltpu.SemaphoreType.DMA,
        ],
    )
    def k(ind_hbm, xt_hbm, xs_hbm, out0_hbm, out1_hbm, outs_hbm,
          ind_v, p0_v, p1_v, idx_v, sign_v, t0_v, t1_v, s_v,
          sem0, sem1, sem2):
        wid = lax.axis_index("s") * 2 + lax.axis_index("c")
        base = wid * bpw

        pltpu.sync_copy(ind_hbm.at[pl.ds(base, bpw)], ind_v)

        def stage(j, carry):
            sl = pl.ds(j * _L, _L)
            v = ind_v[sl]
            w = v >= n_rows
            idx = jnp.where(w, v - n_rows, v)
            p0 = lax.shift_left(
                lax.shift_right_logical(idx, 7), 8) + (idx & (_R - 1))
            idx_v[sl] = idx
            p0_v[sl] = p0
            p1_v[sl] = p0 + _R
            sign_v[sl] = jnp.where(w, jnp.float32(-1.0), jnp.float32(1.0))
            return carry

        lax.fori_loop(0, ch, stage, 0)

        c0 = pltpu.async_copy(xt_hbm.at[p0_v], t0_v, sem0)
        c1 = pltpu.async_copy(xt_hbm.at[p1_v], t1_v, sem1)
        c2 = pltpu.async_copy(xs_hbm.at[idx_v], s_v, sem2)
        c0.wait()

        def smul(j, carry):
            sl = pl.ds(j * _L, _L)
            t0_v[sl] = t0_v[sl] * sign_v[sl]
            return carry

        lax.fori_loop(0, ch, smul, 0)

        pltpu.sync_copy(t0_v, out0_hbm.at[pl.ds(base, bpw)])
        c1.wait()
        pltpu.sync_copy(t1_v, out1_hbm.at[pl.ds(base, bpw)])
        c2.wait()
        pltpu.sync_copy(s_v, outs_hbm.at[pl.ds(base, bpw)])

    return k


def kernel(indices, ds_translation, ds_scale):
    n_rows = ds_translation.shape[0]
    batch = indices.shape[0]
    n_pad = -(-n_rows // 1024) * 1024
    pad_rows = n_pad - n_rows
    n_tiles = n_pad // _R

    xt = jnp.pad(ds_translation, ((0, pad_rows), (0, 0))) \
        .reshape(n_tiles, _R, 2).transpose(0, 2, 1).reshape(-1)
    xs = jnp.pad(ds_scale, ((0, pad_rows), (0, 0))).reshape(-1)

    k = _dataset_params_sc(n_rows, batch)
    t0s, t1, s = k(indices.astype(jnp.int32), xt, xs)

    translation_delta = jnp.stack([t0s, t1, jnp.zeros_like(t0s)], axis=1)
    return (translation_delta, s.reshape(batch, 1))
